# Initial kernel scaffold; baseline (speedup 1.0000x reference)
import jax
import jax.numpy as jnp
from jax.experimental import pallas as pl


def kernel(x, edge_index, W, att_src, att_dst, bias, lin_W, lin_b):
    raise NotImplementedError("write your pallas kernel here")


# speedup vs baseline: 25.3506x; 25.3506x over previous
import functools

import jax
import jax.numpy as jnp
from jax import lax
from jax.experimental import pallas as pl
from jax.experimental.pallas import tpu as pltpu
from jax.experimental.pallas import tpu_sc as plsc

N = 10000
E = 320000
F = 128
E2 = E + N

NCORE = 2
NSUB = 16
NW = NCORE * NSUB
K = 128
CH = -(-E2 // (NW * K))
EPW = CH * K
E2P = EPW * NW
NP = 10240
RPT = NP // NSUB


def _pre_body(x_ref, w_ref, asrc_ref, adst_ref, h_ref, asd_ref, amax_ref):
    hb = jnp.dot(x_ref[...], w_ref[...], preferred_element_type=jnp.float32)
    a_s = jnp.sum(hb * asrc_ref[...], axis=1)
    a_d = jnp.sum(hb * adst_ref[...], axis=1)
    h_ref[...] = hb
    asd_ref[...] = jnp.stack([a_s, a_d], axis=0)
    amax_ref[...] = jnp.full((1, 128), jnp.max(a_s), jnp.float32)


_pre = pl.pallas_call(
    _pre_body,
    out_shape=[
        jax.ShapeDtypeStruct((N, F), jnp.float32),
        jax.ShapeDtypeStruct((2, N), jnp.float32),
        jax.ShapeDtypeStruct((1, 128), jnp.float32),
    ],
)


def _post_body(acc_ref, den_ref, bias_ref, linw_ref, linb_ref, y_ref):
    a = acc_ref[0] + acc_ref[1]
    den = jnp.sum(den_ref[...], axis=0)
    o = a[:N] / (den[:, None] + 1e-16) + bias_ref[...]
    o = jnp.where(o > 0, o, jnp.exp(jnp.minimum(o, 0.0)) - 1.0)
    y_ref[...] = jnp.dot(o, linw_ref[...],
                         preferred_element_type=jnp.float32) + linb_ref[...]


_post = pl.pallas_call(
    _post_body,
    out_shape=jax.ShapeDtypeStruct((N, F), jnp.float32),
)


@functools.partial(
    pl.kernel,
    out_type=[
        jax.ShapeDtypeStruct((NCORE, NP, F), jnp.float32),
        jax.ShapeDtypeStruct((NW, N), jnp.float32),
    ],
    mesh=plsc.VectorSubcoreMesh(core_axis_name="c", subcore_axis_name="s"),
    compiler_params=pltpu.CompilerParams(needs_layout_passes=False),
    scratch_types=[
        pltpu.VMEM((K,), jnp.int32),
        pltpu.VMEM((K,), jnp.int32),
        pltpu.VMEM((K,), jnp.float32),
        pltpu.VMEM((K, F), jnp.float32),
        pltpu.VMEM((N,), jnp.float32),
        pltpu.VMEM((N,), jnp.float32),
        pltpu.VMEM((N,), jnp.float32),
        pltpu.VMEM((16,), jnp.float32),
        pltpu.VMEM_SHARED((NP, F), jnp.float32),
        pltpu.SemaphoreType.DMA,
        pltpu.SemaphoreType.DMA,
    ],
)
def _sc_edge(srcp, dstp, asd, amax, zeros2, zeros1, htab, out, dout,
             sidx, didx, wbuf, rows, astab, adtab, dtab, avec, acc,
             semg, sems):
    cid = lax.axis_index("c")
    sid = lax.axis_index("s")
    wid = cid * NSUB + sid

    pltpu.sync_copy(zeros2.at[pl.ds(sid * RPT, RPT)],
                    acc.at[pl.ds(sid * RPT, RPT)])
    pltpu.sync_copy(zeros1, dtab)
    pltpu.sync_copy(asd.at[0], astab)
    pltpu.sync_copy(asd.at[1], adtab)
    pltpu.sync_copy(amax.at[0, pl.ds(0, 16)], avec)
    plsc.subcore_barrier()

    base0 = wid * EPW

    def chunk_body(t, carry):
        base = base0 + t * K
        pltpu.sync_copy(srcp.at[pl.ds(base, K)], sidx)
        pltpu.sync_copy(dstp.at[pl.ds(base, K)], didx)
        gat = pltpu.async_copy(htab.at[sidx], rows, semg)

        a16 = avec[...]
        for j in range(K // 16):
            s16 = sidx[pl.ds(j * 16, 16)]
            d16 = didx[pl.ds(j * 16, 16)]
            as16 = plsc.load_gather(astab, [s16])
            ad16 = plsc.load_gather(adtab, [d16])
            t1 = as16 + ad16
            u = jnp.maximum(t1, 0.2 * t1)
            c0 = a16 + ad16
            c = jnp.maximum(c0, 0.2 * c0)
            w = jnp.exp(u - c)
            gidx = base + j * 16 + lax.iota(jnp.int32, 16)
            w = jnp.where(gidx < E2, w, 0.0)
            plsc.addupdate_scatter(dtab, [d16], w)
            wbuf[pl.ds(j * 16, 16)] = w

        gat.wait()

        def row_body(r, rc):
            wspl = plsc.load_gather(wbuf, [jnp.full((16,), r, jnp.int32)])
            for v in range(F // 16):
                rows[r, pl.ds(v * 16, 16)] = rows[r, pl.ds(v * 16, 16)] * wspl
            return rc

        lax.fori_loop(0, K, row_body, 0)
        pltpu.async_copy(rows, acc.at[didx], sems, add=True).wait()
        return carry

    lax.fori_loop(0, CH, chunk_body, 0)
    plsc.subcore_barrier()
    pltpu.sync_copy(acc.at[pl.ds(sid * RPT, RPT)],
                    out.at[cid, pl.ds(sid * RPT, RPT)])
    pltpu.sync_copy(dtab, dout.at[wid])


def kernel(x, edge_index, W, att_src, att_dst, bias, lin_W, lin_b):
    n = x.shape[0]
    ar = jnp.arange(n, dtype=edge_index.dtype)
    pad = jnp.zeros((E2P - E2,), edge_index.dtype)
    srcp = jnp.concatenate([edge_index[0], ar, pad])
    dstp = jnp.concatenate([edge_index[1], ar, pad])

    htab, asd, amax = _pre(x, W, att_src.reshape(1, F), att_dst.reshape(1, F))
    zeros2 = jnp.zeros((NP, F), jnp.float32)
    zeros1 = jnp.zeros((N,), jnp.float32)
    acc, den = _sc_edge(srcp, dstp, asd, amax, zeros2, zeros1, htab)
    y = _post(acc, den, bias.reshape(1, F), lin_W, lin_b.reshape(1, F))
    return y


# speedup vs baseline: 37.8787x; 1.4942x over previous
import functools

import jax
import jax.numpy as jnp
from jax import lax
from jax.experimental import pallas as pl
from jax.experimental.pallas import tpu as pltpu
from jax.experimental.pallas import tpu_sc as plsc

N = 10000
E = 320000
F = 128
E2 = E + N

NCORE = 2
NSUB = 16
NW = NCORE * NSUB
K = 64
CH = -(-E2 // (NW * K))
EPW = CH * K
E2P = EPW * NW
NP = 10240
RPT = NP // NSUB

QS = 512.0
QC = 63.9


def _pre_body(x_ref, w_ref, asrc_ref, adst_ref, h_ref, pq_ref, amax_ref):
    hb = jnp.dot(x_ref[...], w_ref[...], preferred_element_type=jnp.float32)
    a_s = jnp.sum(hb * asrc_ref[...], axis=1)
    a_d = jnp.sum(hb * adst_ref[...], axis=1)
    h_ref[...] = hb
    asi = (jnp.clip(a_s, -QC, QC) * QS).astype(jnp.int32)
    adi = (jnp.clip(a_d, -QC, QC) * QS).astype(jnp.int32)
    pq_ref[...] = ((asi << 16) | (adi & 0xFFFF))[None, :]
    amax_ref[...] = jnp.full((1, 128), jnp.max(a_s), jnp.float32)


_pre = pl.pallas_call(
    _pre_body,
    out_shape=[
        jax.ShapeDtypeStruct((N, F), jnp.float32),
        jax.ShapeDtypeStruct((1, N), jnp.int32),
        jax.ShapeDtypeStruct((1, 128), jnp.float32),
    ],
)


def _post_body(acc_ref, den_ref, bias_ref, linw_ref, linb_ref, y_ref):
    a = acc_ref[0] + acc_ref[1]
    den = jnp.sum(den_ref[...], axis=0)
    o = a[:N] / (den[:, None] + 1e-16) + bias_ref[...]
    o = jnp.where(o > 0, o, jnp.exp(jnp.minimum(o, 0.0)) - 1.0)
    y_ref[...] = jnp.dot(o, linw_ref[...],
                         preferred_element_type=jnp.float32) + linb_ref[...]


_post = pl.pallas_call(
    _post_body,
    out_shape=jax.ShapeDtypeStruct((N, F), jnp.float32),
)


@functools.partial(
    pl.kernel,
    out_type=[
        jax.ShapeDtypeStruct((NCORE, NP, F), jnp.float32),
        jax.ShapeDtypeStruct((NW, N), jnp.float32),
    ],
    mesh=plsc.VectorSubcoreMesh(core_axis_name="c", subcore_axis_name="s"),
    compiler_params=pltpu.CompilerParams(needs_layout_passes=False),
    scratch_types=(
        [pltpu.VMEM((K,), jnp.int32)] * 3 +
        [pltpu.VMEM((K,), jnp.int32)] * 3 +
        [pltpu.VMEM((K,), jnp.int32)] * 3 +
        [pltpu.VMEM((K,), jnp.float32)] * 3 +
        [pltpu.VMEM((K, F), jnp.float32)] * 3 +
        [
            pltpu.VMEM((N,), jnp.int32),
            pltpu.VMEM((N,), jnp.float32),
            pltpu.VMEM((16,), jnp.float32),
            pltpu.VMEM_SHARED((NP, F), jnp.float32),
        ] +
        [pltpu.SemaphoreType.DMA] * 3 +
        [pltpu.SemaphoreType.DMA] * 3
    ),
)
def _sc_edge(spd, pq, amax, zeros2, zeros1, htab, out, dout,
             sp0, sp1, sp2, si0, si1, si2, di0, di1, di2,
             wb0, wb1, wb2, ro0, ro1, ro2,
             pqtab, dtab, avec, acc,
             sg0, sg1, sg2, ss0, ss1, ss2):
    spbufs = [sp0, sp1, sp2]
    sidxs = [si0, si1, si2]
    didxs = [di0, di1, di2]
    wbufs = [wb0, wb1, wb2]
    rowss = [ro0, ro1, ro2]
    semgs = [sg0, sg1, sg2]
    semss = [ss0, ss1, ss2]

    cid = lax.axis_index("c")
    sid = lax.axis_index("s")
    wid = cid * NSUB + sid

    pltpu.sync_copy(zeros2.at[pl.ds(sid * RPT, RPT)],
                    acc.at[pl.ds(sid * RPT, RPT)])
    pltpu.sync_copy(zeros1, dtab)
    pltpu.sync_copy(pq.at[0], pqtab)
    pltpu.sync_copy(amax.at[0, pl.ds(0, 16)], avec)
    plsc.subcore_barrier()

    base0 = wid * EPW
    inv_qs = 1.0 / QS

    def issue(t, b):
        base = base0 + t * K
        pltpu.sync_copy(spd.at[pl.ds(base, K)], spbufs[b])
        a16 = avec[...]
        for j in range(K // 16):
            sp16 = spbufs[b][pl.ds(j * 16, 16)]
            s16 = sp16 >> 14
            d16 = sp16 & 16383
            sidxs[b][pl.ds(j * 16, 16)] = s16
            didxs[b][pl.ds(j * 16, 16)] = d16
            ps = plsc.load_gather(pqtab, [s16])
            pd = plsc.load_gather(pqtab, [d16])
            as16 = (ps >> 16).astype(jnp.float32) * inv_qs
            ad16 = ((pd << 16) >> 16).astype(jnp.float32) * inv_qs
            t1 = as16 + ad16
            u = jnp.maximum(t1, 0.2 * t1)
            c0 = a16 + ad16
            c = jnp.maximum(c0, 0.2 * c0)
            w = jnp.exp(u - c)
            gidx = base + j * 16 + lax.iota(jnp.int32, 16)
            w = jnp.where(gidx < E2, w, 0.0)
            plsc.addupdate_scatter(dtab, [d16], w)
            wbufs[b][pl.ds(j * 16, 16)] = w
        pltpu.async_copy(htab.at[sidxs[b]], rowss[b], semgs[b])

    def finish(t, b):
        pltpu.make_async_copy(htab.at[sidxs[b]], rowss[b], semgs[b]).wait()

        def row_body(r, rc):
            wspl = plsc.load_gather(wbufs[b], [jnp.full((16,), r, jnp.int32)])
            for v in range(F // 16):
                rowss[b][r, pl.ds(v * 16, 16)] = (
                    rowss[b][r, pl.ds(v * 16, 16)] * wspl)
            return rc

        lax.fori_loop(0, K, row_body, 0, unroll=2)
        pltpu.async_copy(rowss[b], acc.at[didxs[b]], semss[b], add=True)

    def drain(b):
        pltpu.make_async_copy(rowss[b], acc.at[didxs[b]], semss[b]).wait()

    issue(0, 0)

    def pipe_body(i, carry):
        for b in range(3):
            t = 3 * i + b
            bn = (b + 1) % 3

            @pl.when(t >= 2)
            def _():
                drain(bn)

            @pl.when(t < CH - 1)
            def _():
                issue(t + 1, bn)

            finish(t, b)
        return carry

    lax.fori_loop(0, CH // 3, pipe_body, 0)
    drain((CH - 2) % 3)
    drain((CH - 1) % 3)
    plsc.subcore_barrier()
    pltpu.sync_copy(acc.at[pl.ds(sid * RPT, RPT)],
                    out.at[cid, pl.ds(sid * RPT, RPT)])
    pltpu.sync_copy(dtab, dout.at[wid])


def kernel(x, edge_index, W, att_src, att_dst, bias, lin_W, lin_b):
    n = x.shape[0]
    ar = jnp.arange(n, dtype=edge_index.dtype)
    pad = jnp.zeros((E2P - E2,), edge_index.dtype)
    srcp = jnp.concatenate([edge_index[0], ar, pad])
    dstp = jnp.concatenate([edge_index[1], ar, pad])
    spd = (srcp << 14) | dstp

    htab, pq, amax = _pre(x, W, att_src.reshape(1, F), att_dst.reshape(1, F))
    zeros2 = jnp.zeros((NP, F), jnp.float32)
    zeros1 = jnp.zeros((N,), jnp.float32)
    acc, den = _sc_edge(spd, pq, amax, zeros2, zeros1, htab)
    y = _post(acc, den, bias.reshape(1, F), lin_W, lin_b.reshape(1, F))
    return y


# speedup vs baseline: 46.3853x; 1.2246x over previous
import functools

import jax
import jax.numpy as jnp
from jax import lax
from jax.experimental import pallas as pl
from jax.experimental.pallas import tpu as pltpu
from jax.experimental.pallas import tpu_sc as plsc

N = 10000
E = 320000
F = 128
E2 = E + N

NCORE = 2
NSUB = 16
NW = NCORE * NSUB
K = 64
CH = -(-E2 // (NW * K))
EPW = CH * K
E2P = EPW * NW
NP = 10240
RPT = NP // NSUB

QS = 512.0
QC = 63.9


def _pre_body(x_ref, w_ref, asrc_ref, adst_ref, h_ref, pq_ref, amax_ref):
    hb = jnp.dot(x_ref[...], w_ref[...], preferred_element_type=jnp.float32)
    a_s = jnp.sum(hb * asrc_ref[...], axis=1)
    a_d = jnp.sum(hb * adst_ref[...], axis=1)
    h_ref[...] = hb
    asi = (jnp.clip(a_s, -QC, QC) * QS).astype(jnp.int32)
    adi = (jnp.clip(a_d, -QC, QC) * QS).astype(jnp.int32)
    pq_ref[...] = ((asi << 16) | (adi & 0xFFFF))[None, :]
    amax_ref[...] = jnp.full((1, 128), jnp.max(a_s), jnp.float32)


_pre = pl.pallas_call(
    _pre_body,
    out_shape=[
        jax.ShapeDtypeStruct((N, F), jnp.float32),
        jax.ShapeDtypeStruct((1, N), jnp.int32),
        jax.ShapeDtypeStruct((1, 128), jnp.float32),
    ],
)


def _post_body(acc_ref, den_ref, bias_ref, linw_ref, linb_ref, y_ref):
    a = acc_ref[0] + acc_ref[1]
    den = jnp.sum(den_ref[...], axis=0)
    o = a[:N] / (den[:, None] + 1e-16) + bias_ref[...]
    o = jnp.where(o > 0, o, jnp.exp(jnp.minimum(o, 0.0)) - 1.0)
    y_ref[...] = jnp.dot(o, linw_ref[...],
                         preferred_element_type=jnp.float32) + linb_ref[...]


_post = pl.pallas_call(
    _post_body,
    out_shape=jax.ShapeDtypeStruct((N, F), jnp.float32),
)


@functools.partial(
    pl.kernel,
    out_type=[
        jax.ShapeDtypeStruct((NCORE, NP, F), jnp.float32),
        jax.ShapeDtypeStruct((NW, N), jnp.float32),
    ],
    mesh=plsc.VectorSubcoreMesh(core_axis_name="c", subcore_axis_name="s"),
    compiler_params=pltpu.CompilerParams(needs_layout_passes=False),
    scratch_types=(
        [pltpu.VMEM((K,), jnp.int32)] * 3 +
        [pltpu.VMEM((K,), jnp.int32)] * 3 +
        [pltpu.VMEM((K,), jnp.int32)] * 3 +
        [pltpu.VMEM((K,), jnp.float32)] * 3 +
        [pltpu.VMEM((K, F), jnp.float32)] * 3 +
        [
            pltpu.VMEM((N,), jnp.int32),
            pltpu.VMEM((N,), jnp.float32),
            pltpu.VMEM((16,), jnp.float32),
            pltpu.VMEM_SHARED((NP, F), jnp.float32),
        ] +
        [pltpu.SemaphoreType.DMA] * 3 +
        [pltpu.SemaphoreType.DMA] * 3
    ),
)
def _sc_edge(spd, pq, amax, zeros2, zeros1, htab, out, dout,
             sp0, sp1, sp2, si0, si1, si2, di0, di1, di2,
             wb0, wb1, wb2, ro0, ro1, ro2,
             pqtab, dtab, avec, acc,
             sg0, sg1, sg2, ss0, ss1, ss2):
    spbufs = [sp0, sp1, sp2]
    sidxs = [si0, si1, si2]
    didxs = [di0, di1, di2]
    wbufs = [wb0, wb1, wb2]
    rowss = [ro0, ro1, ro2]
    semgs = [sg0, sg1, sg2]
    semss = [ss0, ss1, ss2]

    cid = lax.axis_index("c")
    sid = lax.axis_index("s")
    wid = cid * NSUB + sid

    pltpu.sync_copy(zeros2.at[pl.ds(sid * RPT, RPT)],
                    acc.at[pl.ds(sid * RPT, RPT)])
    pltpu.sync_copy(zeros1, dtab)
    pltpu.sync_copy(pq.at[0], pqtab)
    pltpu.sync_copy(amax.at[0, pl.ds(0, 16)], avec)
    plsc.subcore_barrier()

    base0 = wid * EPW
    inv_qs = 1.0 / QS

    def issue(t, b):
        base = base0 + t * K
        pltpu.sync_copy(spd.at[pl.ds(base, K)], spbufs[b])
        a16 = avec[...]
        for j in range(K // 16):
            sp16 = spbufs[b][pl.ds(j * 16, 16)]
            s16 = sp16 >> 14
            d16 = sp16 & 16383
            sidxs[b][pl.ds(j * 16, 16)] = s16
            didxs[b][pl.ds(j * 16, 16)] = d16
            ps = plsc.load_gather(pqtab, [s16])
            pd = plsc.load_gather(pqtab, [d16])
            as16 = (ps >> 16).astype(jnp.float32) * inv_qs
            ad16 = ((pd << 16) >> 16).astype(jnp.float32) * inv_qs
            t1 = as16 + ad16
            u = jnp.maximum(t1, 0.2 * t1)
            c0 = a16 + ad16
            c = jnp.maximum(c0, 0.2 * c0)
            w = jnp.exp(u - c)
            gidx = base + j * 16 + lax.iota(jnp.int32, 16)
            w = jnp.where(gidx < E2, w, 0.0)
            plsc.addupdate_scatter(dtab, [d16], w)
            wbufs[b][pl.ds(j * 16, 16)] = w
        pltpu.async_copy(htab.at[sidxs[b]], rowss[b], semgs[b])

    def finish(t, b):
        pltpu.make_async_copy(htab.at[sidxs[b]], rowss[b], semgs[b]).wait()

        def row_body(r, rc):
            wspl = plsc.load_gather(wbufs[b], [jnp.full((16,), r, jnp.int32)])
            for v in range(F // 16):
                rowss[b][r, pl.ds(v * 16, 16)] = (
                    rowss[b][r, pl.ds(v * 16, 16)] * wspl)
            return rc

        lax.fori_loop(0, K, row_body, 0, unroll=2)
        pltpu.async_copy(rowss[b], acc.at[didxs[b]], semss[b], add=True)

    def drain(b):
        pltpu.make_async_copy(rowss[b], acc.at[didxs[b]], semss[b]).wait()

    issue(0, 0)

    def pipe_body(i, carry):
        for b in range(3):
            t = 3 * i + b
            bn = (b + 1) % 3

            @pl.when(t >= 2)
            def _():
                drain(bn)

            @pl.when(t < CH - 1)
            def _():
                issue(t + 1, bn)

            finish(t, b)
        return carry

    lax.fori_loop(0, CH // 3, pipe_body, 0)
    drain((CH - 2) % 3)
    drain((CH - 1) % 3)
    plsc.subcore_barrier()
    pltpu.sync_copy(acc.at[pl.ds(sid * RPT, RPT)],
                    out.at[cid, pl.ds(sid * RPT, RPT)])
    pltpu.sync_copy(dtab, dout.at[wid])


def kernel(x, edge_index, W, att_src, att_dst, bias, lin_W, lin_b):
    n = x.shape[0]
    ar = jnp.arange(n, dtype=edge_index.dtype)
    pad = jnp.arange(E2P - E2, dtype=edge_index.dtype) % n
    srcp = jnp.concatenate([edge_index[0], ar, pad])
    dstp = jnp.concatenate([edge_index[1], ar, pad])
    spd = (srcp << 14) | dstp

    htab, pq, amax = _pre(x, W, att_src.reshape(1, F), att_dst.reshape(1, F))
    zeros2 = jnp.zeros((NP, F), jnp.float32)
    zeros1 = jnp.zeros((N,), jnp.float32)
    acc, den = _sc_edge(spd, pq, amax, zeros2, zeros1, htab)
    y = _post(acc, den, bias.reshape(1, F), lin_W, lin_b.reshape(1, F))
    return y


# speedup vs baseline: 55.0012x; 1.1857x over previous
import functools

import jax
import jax.numpy as jnp
from jax import lax
from jax.experimental import pallas as pl
from jax.experimental.pallas import tpu as pltpu
from jax.experimental.pallas import tpu_sc as plsc

N = 10000
E = 320000
F = 128
E2 = E + N

NCORE = 2
NSUB = 16
NW = NCORE * NSUB
K = 64
CH = -(-E2 // (NW * K))
EPW = CH * K
E2P = EPW * NW
NP = 10240
RPT = NP // NSUB

QS = 512.0
QC = 63.9


def _pre_body(x_ref, w_ref, asrc_ref, adst_ref, h_ref, pq_ref, amax_ref):
    hb = jnp.dot(x_ref[...], w_ref[...], preferred_element_type=jnp.float32)
    a_s = jnp.sum(hb * asrc_ref[...], axis=1)
    a_d = jnp.sum(hb * adst_ref[...], axis=1)
    h_ref[...] = hb
    asi = (jnp.clip(a_s, -QC, QC) * QS).astype(jnp.int32)
    adi = (jnp.clip(a_d, -QC, QC) * QS).astype(jnp.int32)
    pq_ref[...] = ((asi << 16) | (adi & 0xFFFF))[None, :]
    amax_ref[...] = jnp.full((1, 128), jnp.max(a_s), jnp.float32)


_pre = pl.pallas_call(
    _pre_body,
    out_shape=[
        jax.ShapeDtypeStruct((N, F), jnp.float32),
        jax.ShapeDtypeStruct((1, N), jnp.int32),
        jax.ShapeDtypeStruct((1, 128), jnp.float32),
    ],
)


def _post_body(acc_ref, den_ref, bias_ref, linw_ref, linb_ref, y_ref):
    a = acc_ref[0] + acc_ref[1]
    den = jnp.sum(den_ref[...], axis=0)
    o = a[:N] / (den[:, None] + 1e-16) + bias_ref[...]
    o = jnp.where(o > 0, o, jnp.exp(jnp.minimum(o, 0.0)) - 1.0)
    y_ref[...] = jnp.dot(o, linw_ref[...],
                         preferred_element_type=jnp.float32) + linb_ref[...]


_post = pl.pallas_call(
    _post_body,
    out_shape=jax.ShapeDtypeStruct((N, F), jnp.float32),
)


@functools.partial(
    pl.kernel,
    out_type=[
        jax.ShapeDtypeStruct((NCORE, NP, F), jnp.float32),
        jax.ShapeDtypeStruct((NW, N), jnp.float32),
    ],
    mesh=plsc.VectorSubcoreMesh(core_axis_name="c", subcore_axis_name="s"),
    compiler_params=pltpu.CompilerParams(needs_layout_passes=False),
    scratch_types=(
        [pltpu.VMEM((K,), jnp.int32)] * 3 +
        [pltpu.VMEM((K,), jnp.int32)] * 3 +
        [pltpu.VMEM((K,), jnp.int32)] * 3 +
        [pltpu.VMEM((K,), jnp.float32)] * 3 +
        [pltpu.VMEM((K, F), jnp.float32)] * 3 +
        [
            pltpu.VMEM((N,), jnp.int32),
            pltpu.VMEM((N,), jnp.float32),
            pltpu.VMEM((16,), jnp.float32),
            pltpu.VMEM_SHARED((NP, F), jnp.float32),
        ] +
        [pltpu.SemaphoreType.DMA] * 3 +
        [pltpu.SemaphoreType.DMA] * 3 +
        [pltpu.SemaphoreType.DMA] * 3
    ),
)
def _sc_edge(spd, pq, amax, zeros2, zeros1, htab, out, dout,
             sp0, sp1, sp2, si0, si1, si2, di0, di1, di2,
             wb0, wb1, wb2, ro0, ro1, ro2,
             pqtab, dtab, avec, acc,
             sg0, sg1, sg2, ss0, ss1, ss2, sp_g0, sp_g1, sp_g2):
    spbufs = [sp0, sp1, sp2]
    semis = [sp_g0, sp_g1, sp_g2]
    sidxs = [si0, si1, si2]
    didxs = [di0, di1, di2]
    wbufs = [wb0, wb1, wb2]
    rowss = [ro0, ro1, ro2]
    semgs = [sg0, sg1, sg2]
    semss = [ss0, ss1, ss2]

    cid = lax.axis_index("c")
    sid = lax.axis_index("s")
    wid = cid * NSUB + sid

    pltpu.sync_copy(zeros2.at[pl.ds(sid * RPT, RPT)],
                    acc.at[pl.ds(sid * RPT, RPT)])
    pltpu.sync_copy(zeros1, dtab)
    pltpu.sync_copy(pq.at[0], pqtab)
    pltpu.sync_copy(amax.at[0, pl.ds(0, 16)], avec)
    plsc.subcore_barrier()

    base0 = wid * EPW
    inv_qs = 1.0 / QS

    def idxstart(t, b):
        base = base0 + t * K
        pltpu.async_copy(spd.at[pl.ds(base, K)], spbufs[b], semis[b])

    def issue(t, b):
        base = base0 + t * K
        pltpu.make_async_copy(spd.at[pl.ds(base, K)], spbufs[b],
                              semis[b]).wait()
        a16 = avec[...]
        for j in range(K // 16):
            sp16 = spbufs[b][pl.ds(j * 16, 16)]
            s16 = sp16 >> 14
            d16 = sp16 & 16383
            sidxs[b][pl.ds(j * 16, 16)] = s16
            didxs[b][pl.ds(j * 16, 16)] = d16
            ps = plsc.load_gather(pqtab, [s16])
            pd = plsc.load_gather(pqtab, [d16])
            as16 = (ps >> 16).astype(jnp.float32) * inv_qs
            ad16 = ((pd << 16) >> 16).astype(jnp.float32) * inv_qs
            t1 = as16 + ad16
            u = jnp.maximum(t1, 0.2 * t1)
            c0 = a16 + ad16
            c = jnp.maximum(c0, 0.2 * c0)
            w = jnp.exp(u - c)
            gidx = base + j * 16 + lax.iota(jnp.int32, 16)
            w = jnp.where(gidx < E2, w, 0.0)
            plsc.addupdate_scatter(dtab, [d16], w)
            wbufs[b][pl.ds(j * 16, 16)] = w
        pltpu.async_copy(htab.at[sidxs[b]], rowss[b], semgs[b])

    def finish(t, b):
        pltpu.make_async_copy(htab.at[sidxs[b]], rowss[b], semgs[b]).wait()

        def row_body(r, rc):
            wspl = plsc.load_gather(wbufs[b], [jnp.full((16,), r, jnp.int32)])
            for v in range(F // 16):
                rowss[b][r, pl.ds(v * 16, 16)] = (
                    rowss[b][r, pl.ds(v * 16, 16)] * wspl)
            return rc

        lax.fori_loop(0, K, row_body, 0, unroll=2)
        pltpu.async_copy(rowss[b], acc.at[didxs[b]], semss[b], add=True)

    def drain(b):
        pltpu.make_async_copy(rowss[b], acc.at[didxs[b]], semss[b]).wait()

    idxstart(0, 0)
    idxstart(1, 1)
    issue(0, 0)

    def pipe_body(i, carry):
        for b in range(3):
            t = 3 * i + b
            bn = (b + 1) % 3

            @pl.when(t < CH - 2)
            def _():
                idxstart(t + 2, (b + 2) % 3)

            @pl.when(t >= 2)
            def _():
                drain(bn)

            @pl.when(t < CH - 1)
            def _():
                issue(t + 1, bn)

            finish(t, b)
        return carry

    lax.fori_loop(0, CH // 3, pipe_body, 0)
    drain((CH - 2) % 3)
    drain((CH - 1) % 3)
    plsc.subcore_barrier()
    pltpu.sync_copy(acc.at[pl.ds(sid * RPT, RPT)],
                    out.at[cid, pl.ds(sid * RPT, RPT)])
    pltpu.sync_copy(dtab, dout.at[wid])


def kernel(x, edge_index, W, att_src, att_dst, bias, lin_W, lin_b):
    n = x.shape[0]
    ar = jnp.arange(n, dtype=edge_index.dtype)
    pad = jnp.arange(E2P - E2, dtype=edge_index.dtype) % n
    srcp = jnp.concatenate([edge_index[0], ar, pad])
    dstp = jnp.concatenate([edge_index[1], ar, pad])
    spd = (srcp << 14) | dstp

    htab, pq, amax = _pre(x, W, att_src.reshape(1, F), att_dst.reshape(1, F))
    zeros2 = jnp.zeros((NP, F), jnp.float32)
    zeros1 = jnp.zeros((N,), jnp.float32)
    acc, den = _sc_edge(spd, pq, amax, zeros2, zeros1, htab)
    y = _post(acc, den, bias.reshape(1, F), lin_W, lin_b.reshape(1, F))
    return y


# speedup vs baseline: 58.5514x; 1.0645x over previous
import functools

import jax
import jax.numpy as jnp
from jax import lax
from jax.experimental import pallas as pl
from jax.experimental.pallas import tpu as pltpu
from jax.experimental.pallas import tpu_sc as plsc

N = 10000
E = 320000
F = 128
E2 = E + N

NCORE = 2
NSUB = 16
NW = NCORE * NSUB
K = 64
CH = -(-E2 // (NW * K))
EPW = CH * K
E2P = EPW * NW
NP = 10240
RPT = NP // NSUB

QS = 512.0
QC = 63.9


ER = E // F
XR = E2P // F - ER


def _pre_body(x_ref, w_ref, asrc_ref, adst_ref, srcm_ref, dstm_ref,
              h_ref, pq_ref, amax_ref, spd_ref):
    hb = jnp.dot(x_ref[...], w_ref[...], preferred_element_type=jnp.float32)
    a_s = jnp.sum(hb * asrc_ref[...], axis=1)
    a_d = jnp.sum(hb * adst_ref[...], axis=1)
    h_ref[...] = hb
    asi = (jnp.clip(a_s, -QC, QC) * QS).astype(jnp.int32)
    adi = (jnp.clip(a_d, -QC, QC) * QS).astype(jnp.int32)
    pq_ref[...] = ((asi << 16) | (adi & 0xFFFF))[None, :]
    amax_ref[...] = jnp.full((1, 128), jnp.max(a_s), jnp.float32)
    spd_ref[:ER] = (srcm_ref[...] << 14) | dstm_ref[...]
    g = (lax.broadcasted_iota(jnp.int32, (XR, F), 0) * F
         + lax.broadcasted_iota(jnp.int32, (XR, F), 1) + ER * F)
    v = jnp.where(g < E2, g - E, g - E2)
    spd_ref[ER:] = v * ((1 << 14) + 1)


_pre = pl.pallas_call(
    _pre_body,
    out_shape=[
        jax.ShapeDtypeStruct((N, F), jnp.float32),
        jax.ShapeDtypeStruct((1, N), jnp.int32),
        jax.ShapeDtypeStruct((1, 128), jnp.float32),
        jax.ShapeDtypeStruct((ER + XR, F), jnp.int32),
    ],
)


def _post_body(acc_ref, den_ref, bias_ref, linw_ref, linb_ref, y_ref):
    a = acc_ref[0] + acc_ref[1]
    den = jnp.sum(den_ref[...], axis=0)
    o = a[:N] / (den[:, None] + 1e-16) + bias_ref[...]
    o = jnp.where(o > 0, o, jnp.exp(jnp.minimum(o, 0.0)) - 1.0)
    y_ref[...] = jnp.dot(o, linw_ref[...],
                         preferred_element_type=jnp.float32) + linb_ref[...]


_post = pl.pallas_call(
    _post_body,
    out_shape=jax.ShapeDtypeStruct((N, F), jnp.float32),
)


@functools.partial(
    pl.kernel,
    out_type=[
        jax.ShapeDtypeStruct((NCORE, NP, F), jnp.float32),
        jax.ShapeDtypeStruct((NW, N), jnp.float32),
    ],
    mesh=plsc.VectorSubcoreMesh(core_axis_name="c", subcore_axis_name="s"),
    compiler_params=pltpu.CompilerParams(needs_layout_passes=False),
    scratch_types=(
        [pltpu.VMEM((K,), jnp.int32)] * 3 +
        [pltpu.VMEM((K,), jnp.int32)] * 3 +
        [pltpu.VMEM((K,), jnp.int32)] * 3 +
        [pltpu.VMEM((K,), jnp.float32)] * 3 +
        [pltpu.VMEM((K, F), jnp.float32)] * 3 +
        [
            pltpu.VMEM((N,), jnp.int32),
            pltpu.VMEM((N,), jnp.float32),
            pltpu.VMEM((16,), jnp.float32),
            pltpu.VMEM_SHARED((NP, F), jnp.float32),
        ] +
        [pltpu.SemaphoreType.DMA] * 3 +
        [pltpu.SemaphoreType.DMA] * 3 +
        [pltpu.SemaphoreType.DMA] * 3
    ),
)
def _sc_edge(spd, pq, amax, htab, out, dout,
             sp0, sp1, sp2, si0, si1, si2, di0, di1, di2,
             wb0, wb1, wb2, ro0, ro1, ro2,
             pqtab, dtab, avec, acc,
             sg0, sg1, sg2, ss0, ss1, ss2, sp_g0, sp_g1, sp_g2):
    spbufs = [sp0, sp1, sp2]
    semis = [sp_g0, sp_g1, sp_g2]
    sidxs = [si0, si1, si2]
    didxs = [di0, di1, di2]
    wbufs = [wb0, wb1, wb2]
    rowss = [ro0, ro1, ro2]
    semgs = [sg0, sg1, sg2]
    semss = [ss0, ss1, ss2]

    cid = lax.axis_index("c")
    sid = lax.axis_index("s")
    wid = cid * NSUB + sid

    zv = jnp.zeros((16,), jnp.float32)

    def zrow(r, c):
        for v in range(F // 16):
            ro0[r, pl.ds(v * 16, 16)] = zv
        return c

    lax.fori_loop(0, K, zrow, 0, unroll=2)
    for q in range(RPT // K):
        pltpu.sync_copy(ro0, acc.at[pl.ds(sid * RPT + q * K, K)])

    def zden(i, c):
        dtab[pl.ds(i * 16, 16)] = zv
        return c

    lax.fori_loop(0, N // 16, zden, 0, unroll=4)
    pltpu.sync_copy(pq.at[0], pqtab)
    pltpu.sync_copy(amax.at[0, pl.ds(0, 16)], avec)
    plsc.subcore_barrier()

    base0 = wid * EPW
    inv_qs = 1.0 / QS

    def idxstart(t, b):
        base = base0 + t * K
        pltpu.async_copy(spd.at[pl.ds(base, K)], spbufs[b], semis[b])

    def issue(t, b):
        base = base0 + t * K
        pltpu.make_async_copy(spd.at[pl.ds(base, K)], spbufs[b],
                              semis[b]).wait()
        a16 = avec[...]
        for j in range(K // 16):
            sp16 = spbufs[b][pl.ds(j * 16, 16)]
            s16 = sp16 >> 14
            d16 = sp16 & 16383
            sidxs[b][pl.ds(j * 16, 16)] = s16
            didxs[b][pl.ds(j * 16, 16)] = d16
            ps = plsc.load_gather(pqtab, [s16])
            pd = plsc.load_gather(pqtab, [d16])
            as16 = (ps >> 16).astype(jnp.float32) * inv_qs
            ad16 = ((pd << 16) >> 16).astype(jnp.float32) * inv_qs
            t1 = as16 + ad16
            u = jnp.maximum(t1, 0.2 * t1)
            c0 = a16 + ad16
            c = jnp.maximum(c0, 0.2 * c0)
            w = jnp.exp(u - c)
            gidx = base + j * 16 + lax.iota(jnp.int32, 16)
            w = jnp.where(gidx < E2, w, 0.0)
            plsc.addupdate_scatter(dtab, [d16], w)
            wbufs[b][pl.ds(j * 16, 16)] = w
        pltpu.async_copy(htab.at[sidxs[b]], rowss[b], semgs[b])

    def finish(t, b):
        pltpu.make_async_copy(htab.at[sidxs[b]], rowss[b], semgs[b]).wait()

        def row_body(r, rc):
            wspl = plsc.load_gather(wbufs[b], [jnp.full((16,), r, jnp.int32)])
            for v in range(F // 16):
                rowss[b][r, pl.ds(v * 16, 16)] = (
                    rowss[b][r, pl.ds(v * 16, 16)] * wspl)
            return rc

        lax.fori_loop(0, K, row_body, 0, unroll=2)
        pltpu.async_copy(rowss[b], acc.at[didxs[b]], semss[b], add=True)

    def drain(b):
        pltpu.make_async_copy(rowss[b], acc.at[didxs[b]], semss[b]).wait()

    idxstart(0, 0)
    idxstart(1, 1)
    issue(0, 0)

    def pipe_body(i, carry):
        for b in range(3):
            t = 3 * i + b
            bn = (b + 1) % 3

            @pl.when(t < CH - 2)
            def _():
                idxstart(t + 2, (b + 2) % 3)

            @pl.when(t >= 2)
            def _():
                drain(bn)

            @pl.when(t < CH - 1)
            def _():
                issue(t + 1, bn)

            finish(t, b)
        return carry

    lax.fori_loop(0, CH // 3, pipe_body, 0)
    drain((CH - 2) % 3)
    drain((CH - 1) % 3)
    plsc.subcore_barrier()
    pltpu.sync_copy(acc.at[pl.ds(sid * RPT, RPT)],
                    out.at[cid, pl.ds(sid * RPT, RPT)])
    pltpu.sync_copy(dtab, dout.at[wid])


def kernel(x, edge_index, W, att_src, att_dst, bias, lin_W, lin_b):
    srcm = edge_index[0].reshape(ER, F)
    dstm = edge_index[1].reshape(ER, F)
    htab, pq, amax, spd2 = _pre(x, W, att_src.reshape(1, F),
                                att_dst.reshape(1, F), srcm, dstm)
    spd = spd2.reshape(E2P)
    acc, den = _sc_edge(spd, pq, amax, htab)
    y = _post(acc, den, bias.reshape(1, F), lin_W, lin_b.reshape(1, F))
    return y


# speedup vs baseline: 60.6862x; 1.0365x over previous
import functools

import jax
import jax.numpy as jnp
from jax import lax
from jax.experimental import pallas as pl
from jax.experimental.pallas import tpu as pltpu
from jax.experimental.pallas import tpu_sc as plsc

N = 10000
E = 320000
F = 128
E2 = E + N

NCORE = 2
NSUB = 16
NW = NCORE * NSUB
K = 64
CH = -(-E2 // (NW * K))
EPW = CH * K
E2P = EPW * NW
NP = 10240
RPT = NP // NSUB

QS = 512.0
QC = 63.9


ER = E // F
XR = E2P // F - ER


def _pre_body(x_ref, w_ref, asrc_ref, adst_ref, srcm_ref, dstm_ref,
              h_ref, pq_ref, amax_ref, spd_ref):
    hb = jnp.dot(x_ref[...], w_ref[...], preferred_element_type=jnp.float32)
    a_s = jnp.sum(hb * asrc_ref[...], axis=1)
    a_d = jnp.sum(hb * adst_ref[...], axis=1)
    h_ref[...] = hb
    asi = (jnp.clip(a_s, -QC, QC) * QS).astype(jnp.int32)
    adi = (jnp.clip(a_d, -QC, QC) * QS).astype(jnp.int32)
    pq_ref[...] = ((asi << 16) | (adi & 0xFFFF))[None, :]
    amax_ref[...] = jnp.full((1, 128), jnp.max(a_s), jnp.float32)
    spd_ref[:ER] = (srcm_ref[...] << 14) | dstm_ref[...]
    g = (lax.broadcasted_iota(jnp.int32, (XR, F), 0) * F
         + lax.broadcasted_iota(jnp.int32, (XR, F), 1) + ER * F)
    v = jnp.where(g < E2, g - E, g - E2)
    spd_ref[ER:] = v * ((1 << 14) + 1)


_pre = pl.pallas_call(
    _pre_body,
    out_shape=[
        jax.ShapeDtypeStruct((N, F), jnp.float32),
        jax.ShapeDtypeStruct((1, N), jnp.int32),
        jax.ShapeDtypeStruct((1, 128), jnp.float32),
        jax.ShapeDtypeStruct((ER + XR, F), jnp.int32),
    ],
)


def _post_body(acc_ref, den_ref, bias_ref, linw_ref, linb_ref, y_ref):
    a = acc_ref[0] + acc_ref[1]
    den = jnp.sum(den_ref[...], axis=0)
    o = a[:N] / (den[:, None] + 1e-16) + bias_ref[...]
    o = jnp.where(o > 0, o, jnp.exp(jnp.minimum(o, 0.0)) - 1.0)
    y_ref[...] = jnp.dot(o, linw_ref[...],
                         preferred_element_type=jnp.float32) + linb_ref[...]


_post = pl.pallas_call(
    _post_body,
    out_shape=jax.ShapeDtypeStruct((N, F), jnp.float32),
)


@functools.partial(
    pl.kernel,
    out_type=[
        jax.ShapeDtypeStruct((NCORE, NP, F), jnp.float32),
        jax.ShapeDtypeStruct((NW, N), jnp.float32),
    ],
    mesh=plsc.VectorSubcoreMesh(core_axis_name="c", subcore_axis_name="s"),
    compiler_params=pltpu.CompilerParams(needs_layout_passes=False),
    scratch_types=(
        [pltpu.VMEM((K,), jnp.int32)] * 3 +
        [pltpu.VMEM((K,), jnp.int32)] * 3 +
        [pltpu.VMEM((K,), jnp.int32)] * 3 +
        [pltpu.VMEM((K,), jnp.float32)] * 3 +
        [pltpu.VMEM((K, F), jnp.float32)] * 3 +
        [
            pltpu.VMEM((N,), jnp.int32),
            pltpu.VMEM((N,), jnp.float32),
            pltpu.VMEM((16,), jnp.float32),
            pltpu.VMEM_SHARED((NP, F), jnp.float32),
        ] +
        [pltpu.SemaphoreType.DMA] * 3 +
        [pltpu.SemaphoreType.DMA] * 3 +
        [pltpu.SemaphoreType.DMA] * 3
    ),
)
def _sc_edge(spd, pq, amax, htab, out, dout,
             sp0, sp1, sp2, si0, si1, si2, di0, di1, di2,
             wb0, wb1, wb2, ro0, ro1, ro2,
             pqtab, dtab, avec, acc,
             sg0, sg1, sg2, ss0, ss1, ss2, sp_g0, sp_g1, sp_g2):
    spbufs = [sp0, sp1, sp2]
    semis = [sp_g0, sp_g1, sp_g2]
    sidxs = [si0, si1, si2]
    didxs = [di0, di1, di2]
    wbufs = [wb0, wb1, wb2]
    rowss = [ro0, ro1, ro2]
    semgs = [sg0, sg1, sg2]
    semss = [ss0, ss1, ss2]

    cid = lax.axis_index("c")
    sid = lax.axis_index("s")
    wid = cid * NSUB + sid

    zv = jnp.zeros((16,), jnp.float32)

    def zrow(r, c):
        for v in range(F // 16):
            ro0[r, pl.ds(v * 16, 16)] = zv
        return c

    lax.fori_loop(0, K, zrow, 0, unroll=2)
    for q in range(RPT // K):
        pltpu.sync_copy(ro0, acc.at[pl.ds(sid * RPT + q * K, K)])

    def zden(i, c):
        dtab[pl.ds(i * 16, 16)] = zv
        return c

    lax.fori_loop(0, N // 16, zden, 0, unroll=4)
    pltpu.sync_copy(pq.at[0], pqtab)
    pltpu.sync_copy(amax.at[0, pl.ds(0, 16)], avec)
    plsc.subcore_barrier()

    base0 = wid * EPW
    inv_qs = 1.0 / QS

    def idxstart(t, b):
        base = base0 + t * K
        pltpu.async_copy(spd.at[pl.ds(base, K)], spbufs[b], semis[b])

    def issue(t, b):
        base = base0 + t * K
        pltpu.make_async_copy(spd.at[pl.ds(base, K)], spbufs[b],
                              semis[b]).wait()
        for j in range(K // 16):
            sp16 = spbufs[b][pl.ds(j * 16, 16)]
            sidxs[b][pl.ds(j * 16, 16)] = sp16 >> 14
            didxs[b][pl.ds(j * 16, 16)] = sp16 & 16383
        pltpu.async_copy(htab.at[sidxs[b]], rowss[b], semgs[b])
        a16 = avec[...]
        for j in range(K // 16):
            s16 = sidxs[b][pl.ds(j * 16, 16)]
            d16 = didxs[b][pl.ds(j * 16, 16)]
            ps = plsc.load_gather(pqtab, [s16])
            pd = plsc.load_gather(pqtab, [d16])
            as16 = (ps >> 16).astype(jnp.float32) * inv_qs
            ad16 = ((pd << 16) >> 16).astype(jnp.float32) * inv_qs
            t1 = as16 + ad16
            u = jnp.maximum(t1, 0.2 * t1)
            c0 = a16 + ad16
            c = jnp.maximum(c0, 0.2 * c0)
            w = jnp.exp(u - c)
            gidx = base + j * 16 + lax.iota(jnp.int32, 16)
            w = jnp.where(gidx < E2, w, 0.0)
            plsc.addupdate_scatter(dtab, [d16], w)
            wbufs[b][pl.ds(j * 16, 16)] = w

    def finish(t, b):
        pltpu.make_async_copy(htab.at[sidxs[b]], rowss[b], semgs[b]).wait()

        def row_body(r, rc):
            wspl = plsc.load_gather(wbufs[b], [jnp.full((16,), r, jnp.int32)])
            for v in range(F // 16):
                rowss[b][r, pl.ds(v * 16, 16)] = (
                    rowss[b][r, pl.ds(v * 16, 16)] * wspl)
            return rc

        lax.fori_loop(0, K, row_body, 0, unroll=4)
        pltpu.async_copy(rowss[b], acc.at[didxs[b]], semss[b], add=True)

    def drain(b):
        pltpu.make_async_copy(rowss[b], acc.at[didxs[b]], semss[b]).wait()

    idxstart(0, 0)
    idxstart(1, 1)
    issue(0, 0)

    def pipe_body(i, carry):
        for b in range(3):
            t = 3 * i + b
            bn = (b + 1) % 3

            @pl.when(t < CH - 2)
            def _():
                idxstart(t + 2, (b + 2) % 3)

            @pl.when(t >= 2)
            def _():
                drain(bn)

            @pl.when(t < CH - 1)
            def _():
                issue(t + 1, bn)

            finish(t, b)
        return carry

    lax.fori_loop(0, CH // 3, pipe_body, 0)
    drain((CH - 2) % 3)
    drain((CH - 1) % 3)
    plsc.subcore_barrier()
    pltpu.sync_copy(acc.at[pl.ds(sid * RPT, RPT)],
                    out.at[cid, pl.ds(sid * RPT, RPT)])
    pltpu.sync_copy(dtab, dout.at[wid])


def kernel(x, edge_index, W, att_src, att_dst, bias, lin_W, lin_b):
    srcm = edge_index[0].reshape(ER, F)
    dstm = edge_index[1].reshape(ER, F)
    htab, pq, amax, spd2 = _pre(x, W, att_src.reshape(1, F),
                                att_dst.reshape(1, F), srcm, dstm)
    spd = spd2.reshape(E2P)
    acc, den = _sc_edge(spd, pq, amax, htab)
    y = _post(acc, den, bias.reshape(1, F), lin_W, lin_b.reshape(1, F))
    return y
